# baseline (speedup 1.0000x reference)
import functools

import jax
import jax.numpy as jnp
from jax import lax
from jax.experimental import pallas as pl
from jax.experimental.pallas import tpu as pltpu
from jax.experimental.pallas import tpu_sc as plsc

N_NODES = 10000
N_EDGES = 320000

NC, NS = 2, 16
NW = NC * NS
CHUNK = 128
SCAT_CHUNKS = 80
NBUF = 8
TOT_CHUNKS = SCAT_CHUNKS + NBUF
E_PAD = NW * SCAT_CHUNKS * CHUNK
R = 10240
RPT = R // NS

_SELU_SCALE = 1.0507009873554805
_SELU_ALPHA = 1.6732632423543772


def _selu(x):
    return _SELU_SCALE * jnp.where(x > 0, x, _SELU_ALPHA * (jnp.exp(x) - 1.0))


def _make_sc_agg(W):
    mesh = plsc.VectorSubcoreMesh(core_axis_name="c", subcore_axis_name="s")

    @functools.partial(
        pl.kernel,
        out_type=(
            jax.ShapeDtypeStruct((R, W), jnp.float32),
            jax.ShapeDtypeStruct((R, W), jnp.float32),
        ),
        mesh=mesh,
        scratch_types=[
            pltpu.VMEM((TOT_CHUNKS, CHUNK), jnp.int32),
            pltpu.VMEM((TOT_CHUNKS, CHUNK), jnp.int32),
            pltpu.VMEM((NBUF, CHUNK, W), jnp.float32),
            pltpu.VMEM((RPT, W), jnp.float32),
            pltpu.VMEM_SHARED((R, W), jnp.float32),
            [pltpu.SemaphoreType.DMA] * NBUF,
            [pltpu.SemaphoreType.DMA] * NBUF,
        ],
        compiler_params=pltpu.CompilerParams(use_tc_tiling_on_sc=False),
    )
    def sc_agg(src_hbm, dst_hbm, tab_hbm, out0, out1,
               idx_s, idx_d, bufs, obuf, acc, gsems, ssems):
        c = lax.axis_index("c")
        s = lax.axis_index("s")
        w = c * NS + s

        pltpu.sync_copy(src_hbm.at[pl.ds(w * TOT_CHUNKS, TOT_CHUNKS)], idx_s)
        pltpu.sync_copy(dst_hbm.at[pl.ds(w * TOT_CHUNKS, TOT_CHUNKS)], idx_d)

        zv = jnp.zeros((16,), jnp.float32)

        def zrow(i, carry):
            for k in range(W // 16):
                obuf[i, pl.ds(k * 16, 16)] = zv
            return carry

        lax.fori_loop(0, RPT, zrow, 0)
        pltpu.sync_copy(obuf, acc.at[pl.ds(s * RPT, RPT)])
        plsc.subcore_barrier()

        def gwait(b):
            pltpu.make_async_copy(tab_hbm.at[idx_s.at[0]], bufs.at[b], gsems[b]).wait()

        def swait(b):
            pltpu.make_async_copy(bufs.at[b], acc.at[idx_d.at[0]], ssems[b]).wait()

        def gstart(j, b):
            pltpu.async_copy(tab_hbm.at[idx_s.at[j]], bufs.at[b], gsems[b])

        for b in range(NBUF):
            gstart(b, b)

        def body(g, carry):
            j0 = g * NBUF
            for b in range(NBUF):
                gwait(b)
                pltpu.async_copy(bufs.at[b], acc.at[idx_d.at[j0 + b]], ssems[b], add=True)
            for b in range(NBUF):
                swait(b)
                gstart(j0 + NBUF + b, b)
            return carry

        lax.fori_loop(0, SCAT_CHUNKS // NBUF, body, 0)

        for b in range(NBUF):
            gwait(b)

        plsc.subcore_barrier()

        pltpu.sync_copy(acc.at[pl.ds(s * RPT, RPT)], obuf)

        @pl.when(c == 0)
        def _():
            pltpu.sync_copy(obuf, out0.at[pl.ds(s * RPT, RPT)])

        @pl.when(c == 1)
        def _():
            pltpu.sync_copy(obuf, out1.at[pl.ds(s * RPT, RPT)])

    return sc_agg


_sc_agg32 = _make_sc_agg(32)
_sc_agg16 = _make_sc_agg(16)


_BLK = 2000
_GRID = N_NODES // _BLK


def _row_block(width):
    return pl.BlockSpec((_BLK, width), lambda i: (i, 0))


def _full_block(shape):
    return pl.BlockSpec(shape, lambda i: (0,) * len(shape))


def _enc_body(x_ref, w0, b0, w1, b1, w2, b2, out_ref):
    h = _selu(jnp.dot(x_ref[...], w0[...], preferred_element_type=jnp.float32) + b0[...])
    h = _selu(jnp.dot(h, w1[...], preferred_element_type=jnp.float32) + b1[...])
    h = _selu(jnp.dot(h, w2[...], preferred_element_type=jnp.float32) + b2[...])
    ones = jnp.ones((_BLK, 1), jnp.float32)
    zeros = jnp.zeros((_BLK, 15), jnp.float32)
    out_ref[...] = jnp.concatenate([h, ones, zeros], axis=1)


def _mid_body(p0, p1, haug, wl1, wr1, b1, wl2, wr2, b2,
              ptab, h1r, rinv_out):
    ssum = p0[...] + p1[...]
    cnt = ssum[:, 16:17]
    rinv = 1.0 / jnp.maximum(cnt, 1.0)
    agg0 = ssum[:, 0:16] * rinv
    h0 = haug[:, 0:16]
    h1 = (jnp.dot(agg0, wl1[...], preferred_element_type=jnp.float32)
          + jnp.dot(h0, wr1[...], preferred_element_type=jnp.float32) + b1[...])
    ptab[...] = jnp.dot(h1, wl2[...], preferred_element_type=jnp.float32)
    h1r[...] = jnp.dot(h1, wr2[...], preferred_element_type=jnp.float32) + b2[...]
    rinv_out[...] = jnp.broadcast_to(rinv, (_BLK, 16))


def _dec_body(q0, q1, h1r, rinv, w0, b0, w1, b1, w2, b2, out_ref):
    h2 = (q0[...] + q1[...]) * rinv[...] + h1r[...]
    h = _selu(jnp.dot(h2, w0[...], preferred_element_type=jnp.float32) + b0[...])
    h = _selu(jnp.dot(h, w1[...], preferred_element_type=jnp.float32) + b1[...])
    h = _selu(jnp.dot(h, w2[...], preferred_element_type=jnp.float32) + b2[...])
    out_ref[...] = h


def _fold(W, b, g, be):
    Wf = (W * g[:, None]).T
    bf = (b * g + be)[None, :]
    return Wf, bf


def kernel(x, edge_index,
           enc0_W, enc0_b, enc0_g, enc0_be,
           enc1_W, enc1_b, enc1_g, enc1_be,
           enc2_W, enc2_b, enc2_g, enc2_be,
           si_Wl, si_Wr, si_b,
           so_Wl, so_Wr, so_b,
           dec0_W, dec0_b, dec0_g, dec0_be,
           dec1_W, dec1_b, dec1_g, dec1_be,
           dec2_W, dec2_b, dec2_g, dec2_be):
    e0w, e0b = _fold(enc0_W, enc0_b, enc0_g, enc0_be)
    e1w, e1b = _fold(enc1_W, enc1_b, enc1_g, enc1_be)
    e2w, e2b = _fold(enc2_W, enc2_b, enc2_g, enc2_be)
    d0w, d0b = _fold(dec0_W, dec0_b, dec0_g, dec0_be)
    d1w, d1b = _fold(dec1_W, dec1_b, dec1_g, dec1_be)
    d2w, d2b = _fold(dec2_W, dec2_b, dec2_g, dec2_be)
    wl1, wr1, b1 = si_Wl.T, si_Wr.T, si_b[None, :]
    wl2, wr2, b2 = so_Wl.T, so_Wr.T, so_b[None, :]

    src, dst = edge_index[0], edge_index[1]
    src_p = jnp.concatenate([src, jnp.zeros((E_PAD - N_EDGES,), jnp.int32)])
    dst_p = jnp.concatenate([dst, jnp.full((E_PAD - N_EDGES,), N_NODES, jnp.int32)])
    tail = TOT_CHUNKS - SCAT_CHUNKS
    src_g = jnp.concatenate(
        [src_p.reshape(NW, SCAT_CHUNKS, CHUNK),
         jnp.zeros((NW, tail, CHUNK), jnp.int32)], axis=1).reshape(NW * TOT_CHUNKS, CHUNK)
    dst_g = jnp.concatenate(
        [dst_p.reshape(NW, SCAT_CHUNKS, CHUNK),
         jnp.full((NW, tail, CHUNK), N_NODES, jnp.int32)], axis=1).reshape(NW * TOT_CHUNKS, CHUNK)

    h_aug = pl.pallas_call(
        _enc_body,
        grid=(_GRID,),
        in_specs=[_row_block(128),
                  _full_block((128, 32)), _full_block((1, 32)),
                  _full_block((32, 32)), _full_block((1, 32)),
                  _full_block((32, 16)), _full_block((1, 16))],
        out_specs=_row_block(32),
        out_shape=jax.ShapeDtypeStruct((N_NODES, 32), jnp.float32),
    )(x, e0w, e0b, e1w, e1b, e2w, e2b)

    p0, p1 = _sc_agg32(src_g, dst_g, h_aug)

    ptab, h1r, rinv = pl.pallas_call(
        _mid_body,
        grid=(_GRID,),
        in_specs=[_row_block(32), _row_block(32), _row_block(32),
                  _full_block((16, 32)), _full_block((16, 32)), _full_block((1, 32)),
                  _full_block((32, 16)), _full_block((32, 16)), _full_block((1, 16))],
        out_specs=[_row_block(16), _row_block(16), _row_block(16)],
        out_shape=[jax.ShapeDtypeStruct((N_NODES, 16), jnp.float32),
                   jax.ShapeDtypeStruct((N_NODES, 16), jnp.float32),
                   jax.ShapeDtypeStruct((N_NODES, 16), jnp.float32)],
    )(p0, p1, h_aug, wl1, wr1, b1, wl2, wr2, b2)

    q0, q1 = _sc_agg16(src_g, dst_g, ptab)

    out = pl.pallas_call(
        _dec_body,
        grid=(_GRID,),
        in_specs=[_row_block(16), _row_block(16), _row_block(16), _row_block(16),
                  _full_block((16, 32)), _full_block((1, 32)),
                  _full_block((32, 32)), _full_block((1, 32)),
                  _full_block((32, 128)), _full_block((1, 128))],
        out_specs=_row_block(128),
        out_shape=jax.ShapeDtypeStruct((N_NODES, 128), jnp.float32),
    )(q0, q1, h1r, rinv, d0w, d0b, d1w, d1b, d2w, d2b)

    return out


# speedup vs baseline: 2.0919x; 2.0919x over previous
import functools

import jax
import jax.numpy as jnp
from jax import lax
from jax.experimental import pallas as pl
from jax.experimental.pallas import tpu as pltpu
from jax.experimental.pallas import tpu_sc as plsc

N_NODES = 10000
N_EDGES = 320000

NC, NS = 2, 16
NW = NC * NS
CHUNK = 128
SCAT_CHUNKS = 80
TOT_CHUNKS = 88
E_PAD = NW * SCAT_CHUNKS * CHUNK
R = 10240
RPT = R // NS
CNT_ROWS = R // 16

_SELU_SCALE = 1.0507009873554805
_SELU_ALPHA = 1.6732632423543772


def _selu(x):
    return _SELU_SCALE * jnp.where(x > 0, x, _SELU_ALPHA * (jnp.exp(x) - 1.0))


def _make_sc_agg():
    W = 16
    mesh = plsc.VectorSubcoreMesh(core_axis_name="c", subcore_axis_name="s")

    @functools.partial(
        pl.kernel,
        out_type=(
            jax.ShapeDtypeStruct((R, W), jnp.float32),
            jax.ShapeDtypeStruct((R, W), jnp.float32),
        ),
        mesh=mesh,
        scratch_types=[
            pltpu.VMEM((TOT_CHUNKS, CHUNK), jnp.int32),
            pltpu.VMEM((TOT_CHUNKS, CHUNK), jnp.int32),
            pltpu.VMEM((CHUNK, W), jnp.float32),
            pltpu.VMEM((CHUNK, W), jnp.float32),
            pltpu.VMEM((RPT, W), jnp.float32),
            pltpu.VMEM_SHARED((R, W), jnp.float32),
            pltpu.SemaphoreType.DMA,
            pltpu.SemaphoreType.DMA,
        ],
        compiler_params=pltpu.CompilerParams(use_tc_tiling_on_sc=False),
    )
    def sc_agg(src_hbm, dst_hbm, tab_hbm, out0, out1,
               idx_s, idx_d, buf0, buf1, obuf, acc, sem0, sem1):
        c = lax.axis_index("c")
        s = lax.axis_index("s")
        w = c * NS + s

        pltpu.sync_copy(src_hbm.at[pl.ds(w * TOT_CHUNKS, TOT_CHUNKS)], idx_s)
        pltpu.sync_copy(dst_hbm.at[pl.ds(w * TOT_CHUNKS, TOT_CHUNKS)], idx_d)

        zv = jnp.zeros((16,), jnp.float32)

        def zrow(i, carry):
            obuf[i, :] = zv
            return carry

        lax.fori_loop(0, RPT, zrow, 0)
        pltpu.sync_copy(obuf, acc.at[pl.ds(s * RPT, RPT)])
        plsc.subcore_barrier()

        def gstart(j, buf, sem):
            return pltpu.async_copy(tab_hbm.at[idx_s.at[j]], buf, sem)

        gstart(0, buf0, sem0)
        gstart(1, buf1, sem1)

        def body(jj, carry):
            j = jj * 2
            pltpu.make_async_copy(tab_hbm.at[idx_s.at[j]], buf0, sem0).wait()
            pltpu.sync_copy(buf0, acc.at[idx_d.at[j]], add=True)
            gstart(j + 2, buf0, sem0)
            pltpu.make_async_copy(tab_hbm.at[idx_s.at[j + 1]], buf1, sem1).wait()
            pltpu.sync_copy(buf1, acc.at[idx_d.at[j + 1]], add=True)
            gstart(j + 3, buf1, sem1)
            return carry

        lax.fori_loop(0, SCAT_CHUNKS // 2, body, 0)

        pltpu.make_async_copy(tab_hbm.at[idx_s.at[0]], buf0, sem0).wait()
        pltpu.make_async_copy(tab_hbm.at[idx_s.at[0]], buf1, sem1).wait()

        plsc.subcore_barrier()

        pltpu.sync_copy(acc.at[pl.ds(s * RPT, RPT)], obuf)

        @pl.when(c == 0)
        def _():
            pltpu.sync_copy(obuf, out0.at[pl.ds(s * RPT, RPT)])

        @pl.when(c == 1)
        def _():
            pltpu.sync_copy(obuf, out1.at[pl.ds(s * RPT, RPT)])

    return sc_agg


_sc_agg16 = _make_sc_agg()

_WBLK = TOT_CHUNKS * CHUNK
_WCNT = SCAT_CHUNKS * CHUNK


def _make_sc_count():
    mesh = plsc.VectorSubcoreMesh(core_axis_name="c", subcore_axis_name="s")

    @functools.partial(
        pl.kernel,
        out_type=(
            jax.ShapeDtypeStruct((R,), jnp.float32),
            jax.ShapeDtypeStruct((R,), jnp.float32),
        ),
        mesh=mesh,
        scratch_types=[
            pltpu.VMEM((_WBLK,), jnp.int32),
            pltpu.VMEM((R,), jnp.float32),
            pltpu.VMEM((RPT,), jnp.float32),
            pltpu.VMEM_SHARED((NS * R,), jnp.float32),
        ],
        compiler_params=pltpu.CompilerParams(use_tc_tiling_on_sc=False,
                                             needs_layout_passes=False),
    )
    def sc_count(dst_hbm, oc0, oc1, idx, cnt, cseg, acc_c):
        c = lax.axis_index("c")
        s = lax.axis_index("s")
        w = c * NS + s
        pltpu.sync_copy(dst_hbm.at[pl.ds(w * _WBLK, _WBLK)], idx)

        zv = jnp.zeros((16,), jnp.float32)
        ov = jnp.ones((16,), jnp.float32)

        def zrow(i, carry):
            cnt[pl.ds(i * 16, 16)] = zv
            return carry

        lax.fori_loop(0, R // 16, zrow, 0)

        def crow(i, carry):
            dv = idx[pl.ds(i * 16, 16)]
            plsc.addupdate_scatter(cnt, [dv], ov)
            return carry

        lax.fori_loop(0, _WCNT // 16, crow, 0)
        pltpu.sync_copy(cnt, acc_c.at[pl.ds(s * R, R)])
        plsc.subcore_barrier()

        def gslice(t, carry):
            pltpu.sync_copy(acc_c.at[pl.ds(t * R + s * RPT, RPT)],
                            cnt.at[pl.ds(t * RPT, RPT)])
            return carry

        lax.fori_loop(0, NS, gslice, 0)

        def redrow(v, carry):
            a = cnt[pl.ds(v * 16, 16)]
            for t in range(1, NS):
                a = a + cnt[pl.ds(t * RPT + v * 16, 16)]
            cseg[pl.ds(v * 16, 16)] = a
            return carry

        lax.fori_loop(0, RPT // 16, redrow, 0)

        @pl.when(c == 0)
        def _():
            pltpu.sync_copy(cseg, oc0.at[pl.ds(s * RPT, RPT)])

        @pl.when(c == 1)
        def _():
            pltpu.sync_copy(cseg, oc1.at[pl.ds(s * RPT, RPT)])

    return sc_count


_sc_count = _make_sc_count()


_BLK = 2000
_GRID = N_NODES // _BLK


def _row_block(width):
    return pl.BlockSpec((_BLK, width), lambda i: (i, 0))


def _full_block(shape):
    return pl.BlockSpec(shape, lambda i: (0,) * len(shape))


def _enc_body(x_ref, w0, b0, w1, b1, w2, b2, out_ref):
    h = _selu(jnp.dot(x_ref[...], w0[...], preferred_element_type=jnp.float32) + b0[...])
    h = _selu(jnp.dot(h, w1[...], preferred_element_type=jnp.float32) + b1[...])
    out_ref[...] = _selu(jnp.dot(h, w2[...], preferred_element_type=jnp.float32) + b2[...])


def _mid_body(p0, p1, c0, c1, h0ref, wl1, wr1, b1, wl2, wr2, b2,
              ptab, h1r, rinv_out):
    ssum = p0[...] + p1[...]
    cnt = c0[...] + c1[...]
    rinv = 1.0 / jnp.maximum(cnt, 1.0)
    agg0 = ssum * rinv
    h0 = h0ref[...]
    h1 = (jnp.dot(agg0, wl1[...], preferred_element_type=jnp.float32)
          + jnp.dot(h0, wr1[...], preferred_element_type=jnp.float32) + b1[...])
    ptab[...] = jnp.dot(h1, wl2[...], preferred_element_type=jnp.float32)
    h1r[...] = jnp.dot(h1, wr2[...], preferred_element_type=jnp.float32) + b2[...]
    rinv_out[...] = jnp.broadcast_to(rinv, (_BLK, 16))


def _dec_body(q0, q1, h1r, rinv, w0, b0, w1, b1, w2, b2, out_ref):
    h2 = (q0[...] + q1[...]) * rinv[...] + h1r[...]
    h = _selu(jnp.dot(h2, w0[...], preferred_element_type=jnp.float32) + b0[...])
    h = _selu(jnp.dot(h, w1[...], preferred_element_type=jnp.float32) + b1[...])
    h = _selu(jnp.dot(h, w2[...], preferred_element_type=jnp.float32) + b2[...])
    out_ref[...] = h


def _fold(W, b, g, be):
    Wf = (W * g[:, None]).T
    bf = (b * g + be)[None, :]
    return Wf, bf


def kernel(x, edge_index,
           enc0_W, enc0_b, enc0_g, enc0_be,
           enc1_W, enc1_b, enc1_g, enc1_be,
           enc2_W, enc2_b, enc2_g, enc2_be,
           si_Wl, si_Wr, si_b,
           so_Wl, so_Wr, so_b,
           dec0_W, dec0_b, dec0_g, dec0_be,
           dec1_W, dec1_b, dec1_g, dec1_be,
           dec2_W, dec2_b, dec2_g, dec2_be):
    e0w, e0b = _fold(enc0_W, enc0_b, enc0_g, enc0_be)
    e1w, e1b = _fold(enc1_W, enc1_b, enc1_g, enc1_be)
    e2w, e2b = _fold(enc2_W, enc2_b, enc2_g, enc2_be)
    d0w, d0b = _fold(dec0_W, dec0_b, dec0_g, dec0_be)
    d1w, d1b = _fold(dec1_W, dec1_b, dec1_g, dec1_be)
    d2w, d2b = _fold(dec2_W, dec2_b, dec2_g, dec2_be)
    wl1, wr1, b1 = si_Wl.T, si_Wr.T, si_b[None, :]
    wl2, wr2, b2 = so_Wl.T, so_Wr.T, so_b[None, :]

    src, dst = edge_index[0], edge_index[1]
    src_p = jnp.concatenate([src, jnp.zeros((E_PAD - N_EDGES,), jnp.int32)])
    dst_p = jnp.concatenate([dst, jnp.full((E_PAD - N_EDGES,), N_NODES, jnp.int32)])
    tail = TOT_CHUNKS - SCAT_CHUNKS
    src_g = jnp.concatenate(
        [src_p.reshape(NW, SCAT_CHUNKS, CHUNK),
         jnp.zeros((NW, tail, CHUNK), jnp.int32)], axis=1).reshape(NW * TOT_CHUNKS, CHUNK)
    dst_g = jnp.concatenate(
        [dst_p.reshape(NW, SCAT_CHUNKS, CHUNK),
         jnp.full((NW, tail, CHUNK), N_NODES, jnp.int32)], axis=1).reshape(NW * TOT_CHUNKS, CHUNK)

    h0 = pl.pallas_call(
        _enc_body,
        grid=(_GRID,),
        in_specs=[_row_block(128),
                  _full_block((128, 32)), _full_block((1, 32)),
                  _full_block((32, 32)), _full_block((1, 32)),
                  _full_block((32, 16)), _full_block((1, 16))],
        out_specs=_row_block(16),
        out_shape=jax.ShapeDtypeStruct((N_NODES, 16), jnp.float32),
    )(x, e0w, e0b, e1w, e1b, e2w, e2b)

    oc0, oc1 = _sc_count(dst_g.reshape(-1))
    c0 = oc0.reshape(R, 1)
    c1 = oc1.reshape(R, 1)

    p0, p1 = _sc_agg16(src_g, dst_g, h0)

    ptab, h1r, rinv = pl.pallas_call(
        _mid_body,
        grid=(_GRID,),
        in_specs=[_row_block(16), _row_block(16),
                  _row_block(1), _row_block(1), _row_block(16),
                  _full_block((16, 32)), _full_block((16, 32)), _full_block((1, 32)),
                  _full_block((32, 16)), _full_block((32, 16)), _full_block((1, 16))],
        out_specs=[_row_block(16), _row_block(16), _row_block(16)],
        out_shape=[jax.ShapeDtypeStruct((N_NODES, 16), jnp.float32),
                   jax.ShapeDtypeStruct((N_NODES, 16), jnp.float32),
                   jax.ShapeDtypeStruct((N_NODES, 16), jnp.float32)],
    )(p0, p1, c0, c1, h0, wl1, wr1, b1, wl2, wr2, b2)

    q0, q1 = _sc_agg16(src_g, dst_g, ptab)

    out = pl.pallas_call(
        _dec_body,
        grid=(_GRID,),
        in_specs=[_row_block(16), _row_block(16), _row_block(16), _row_block(16),
                  _full_block((16, 32)), _full_block((1, 32)),
                  _full_block((32, 32)), _full_block((1, 32)),
                  _full_block((32, 128)), _full_block((1, 128))],
        out_specs=_row_block(128),
        out_shape=jax.ShapeDtypeStruct((N_NODES, 128), jnp.float32),
    )(q0, q1, h1r, rinv, d0w, d0b, d1w, d1b, d2w, d2b)

    return out


# speedup vs baseline: 2.1882x; 1.0460x over previous
import functools

import jax
import jax.numpy as jnp
from jax import lax
from jax.experimental import pallas as pl
from jax.experimental.pallas import tpu as pltpu
from jax.experimental.pallas import tpu_sc as plsc

N_NODES = 10000
N_EDGES = 320000

NC, NS = 2, 16
NW = NC * NS
CHUNK = 128
SCAT_CHUNKS = 80
TOT_CHUNKS = 88
E_PAD = NW * SCAT_CHUNKS * CHUNK
R = 10240
RPT = R // NS
_WBLK = TOT_CHUNKS * CHUNK
_WCNT = SCAT_CHUNKS * CHUNK

_SELU_SCALE = 1.0507009873554805
_SELU_ALPHA = 1.6732632423543772


def _selu(x):
    return _SELU_SCALE * jnp.where(x > 0, x, _SELU_ALPHA * (jnp.exp(x) - 1.0))


def _make_sc_agg(with_counts):
    W = 16
    mesh = plsc.VectorSubcoreMesh(core_axis_name="c", subcore_axis_name="s")
    out_types = [
        jax.ShapeDtypeStruct((R, W), jnp.float32),
        jax.ShapeDtypeStruct((R, W), jnp.float32),
    ]
    scratch = [
        pltpu.VMEM((TOT_CHUNKS, CHUNK), jnp.int32),
        pltpu.VMEM((TOT_CHUNKS, CHUNK), jnp.int32),
        pltpu.VMEM((CHUNK, W), jnp.float32),
        pltpu.VMEM((CHUNK, W), jnp.float32),
        pltpu.VMEM((RPT, W), jnp.float32),
        pltpu.VMEM_SHARED((R, W), jnp.float32),
        pltpu.SemaphoreType.DMA,
        pltpu.SemaphoreType.DMA,
    ]
    if with_counts:
        out_types += [
            jax.ShapeDtypeStruct((R,), jnp.float32),
            jax.ShapeDtypeStruct((R,), jnp.float32),
        ]
        scratch += [
            pltpu.VMEM((_WCNT,), jnp.int32),
            pltpu.VMEM((R,), jnp.float32),
            pltpu.VMEM((RPT,), jnp.float32),
            pltpu.VMEM_SHARED((NS * R,), jnp.float32),
        ]

    @functools.partial(
        pl.kernel,
        out_type=tuple(out_types),
        mesh=mesh,
        scratch_types=scratch,
        compiler_params=pltpu.CompilerParams(use_tc_tiling_on_sc=False,
                                             needs_layout_passes=not with_counts),
    )
    def sc_agg(src_hbm, dst_hbm, dstf_hbm, zeros_hbm, tab_hbm, *rest):
        if with_counts:
            (out0, out1, oc0, oc1,
             idx_s, idx_d, buf0, buf1, obuf, acc, sem0, sem1,
             idx_f, cnt, cseg, acc_c) = rest
        else:
            (out0, out1,
             idx_s, idx_d, buf0, buf1, obuf, acc, sem0, sem1) = rest
        c = lax.axis_index("c")
        s = lax.axis_index("s")
        w = c * NS + s

        pltpu.sync_copy(src_hbm.at[pl.ds(w * TOT_CHUNKS, TOT_CHUNKS)], idx_s)
        pltpu.sync_copy(dst_hbm.at[pl.ds(w * TOT_CHUNKS, TOT_CHUNKS)], idx_d)

        pltpu.sync_copy(zeros_hbm, obuf)
        pltpu.sync_copy(obuf, acc.at[pl.ds(s * RPT, RPT)])

        zv = jnp.zeros((16,), jnp.float32)
        ov = jnp.ones((16,), jnp.float32)
        if with_counts:
            pltpu.sync_copy(dstf_hbm.at[w, pl.ds(0, _WCNT)], idx_f)

            def zcrow(i, carry):
                cnt[pl.ds(i * 16, 16)] = zv
                return carry

            lax.fori_loop(0, R // 16, zcrow, 0)
        plsc.subcore_barrier()

        def gstart(j, buf, sem):
            return pltpu.async_copy(tab_hbm.at[idx_s.at[j]], buf, sem)

        gstart(0, buf0, sem0)
        gstart(1, buf1, sem1)

        def body(jj, carry):
            j = jj * 2
            pltpu.make_async_copy(tab_hbm.at[idx_s.at[j]], buf0, sem0).wait()
            pltpu.sync_copy(buf0, acc.at[idx_d.at[j]], add=True)
            gstart(j + 2, buf0, sem0)
            if with_counts:
                for k in range(2 * CHUNK // 16):
                    dv = idx_f[pl.ds(jj * 2 * CHUNK + k * 16, 16)]
                    plsc.addupdate_scatter(cnt, [dv], ov)
            pltpu.make_async_copy(tab_hbm.at[idx_s.at[j + 1]], buf1, sem1).wait()
            pltpu.sync_copy(buf1, acc.at[idx_d.at[j + 1]], add=True)
            gstart(j + 3, buf1, sem1)
            return carry

        lax.fori_loop(0, SCAT_CHUNKS // 2, body, 0)

        pltpu.make_async_copy(tab_hbm.at[idx_s.at[0]], buf0, sem0).wait()
        pltpu.make_async_copy(tab_hbm.at[idx_s.at[0]], buf1, sem1).wait()

        if with_counts:
            pltpu.sync_copy(cnt, acc_c.at[pl.ds(s * R, R)])

        plsc.subcore_barrier()

        pltpu.sync_copy(acc.at[pl.ds(s * RPT, RPT)], obuf)

        @pl.when(c == 0)
        def _():
            pltpu.sync_copy(obuf, out0.at[pl.ds(s * RPT, RPT)])

        @pl.when(c == 1)
        def _():
            pltpu.sync_copy(obuf, out1.at[pl.ds(s * RPT, RPT)])

        if with_counts:
            def gslice(t, carry):
                pltpu.sync_copy(acc_c.at[pl.ds(t * R + s * RPT, RPT)],
                                cnt.at[pl.ds(t * RPT, RPT)])
                return carry

            lax.fori_loop(0, NS, gslice, 0)

            def redrow(v, carry):
                a = cnt[pl.ds(v * 16, 16)]
                for t in range(1, NS):
                    a = a + cnt[pl.ds(t * RPT + v * 16, 16)]
                cseg[pl.ds(v * 16, 16)] = a
                return carry

            lax.fori_loop(0, RPT // 16, redrow, 0)

            @pl.when(c == 0)
            def _():
                pltpu.sync_copy(cseg, oc0.at[pl.ds(s * RPT, RPT)])

            @pl.when(c == 1)
            def _():
                pltpu.sync_copy(cseg, oc1.at[pl.ds(s * RPT, RPT)])

    return sc_agg


_sc_agg_cnt = _make_sc_agg(True)
_sc_agg16 = _make_sc_agg(False)


_BLK = 2000
_GRID = N_NODES // _BLK


def _row_block(width):
    return pl.BlockSpec((_BLK, width), lambda i: (i, 0))


def _full_block(shape):
    return pl.BlockSpec(shape, lambda i: (0,) * len(shape))


def _enc_body(x_ref, w0, b0, w1, b1, w2, b2, out_ref):
    h = _selu(jnp.dot(x_ref[...], w0[...], preferred_element_type=jnp.float32) + b0[...])
    h = _selu(jnp.dot(h, w1[...], preferred_element_type=jnp.float32) + b1[...])
    out_ref[...] = _selu(jnp.dot(h, w2[...], preferred_element_type=jnp.float32) + b2[...])


def _mid_body(p0, p1, c0, c1, h0ref, wl1, wr1, b1, wl2, wr2, b2,
              ptab, h1r, rinv_out):
    ssum = p0[...] + p1[...]
    cnt = c0[...] + c1[...]
    rinv = 1.0 / jnp.maximum(cnt, 1.0)
    agg0 = ssum * rinv
    h0 = h0ref[...]
    h1 = (jnp.dot(agg0, wl1[...], preferred_element_type=jnp.float32)
          + jnp.dot(h0, wr1[...], preferred_element_type=jnp.float32) + b1[...])
    ptab[...] = jnp.dot(h1, wl2[...], preferred_element_type=jnp.float32)
    h1r[...] = jnp.dot(h1, wr2[...], preferred_element_type=jnp.float32) + b2[...]
    rinv_out[...] = jnp.broadcast_to(rinv, (_BLK, 16))


def _dec_body(q0, q1, h1r, rinv, w0, b0, w1, b1, w2, b2, out_ref):
    h2 = (q0[...] + q1[...]) * rinv[...] + h1r[...]
    h = _selu(jnp.dot(h2, w0[...], preferred_element_type=jnp.float32) + b0[...])
    h = _selu(jnp.dot(h, w1[...], preferred_element_type=jnp.float32) + b1[...])
    h = _selu(jnp.dot(h, w2[...], preferred_element_type=jnp.float32) + b2[...])
    out_ref[...] = h


def _fold(W, b, g, be):
    Wf = (W * g[:, None]).T
    bf = (b * g + be)[None, :]
    return Wf, bf


def kernel(x, edge_index,
           enc0_W, enc0_b, enc0_g, enc0_be,
           enc1_W, enc1_b, enc1_g, enc1_be,
           enc2_W, enc2_b, enc2_g, enc2_be,
           si_Wl, si_Wr, si_b,
           so_Wl, so_Wr, so_b,
           dec0_W, dec0_b, dec0_g, dec0_be,
           dec1_W, dec1_b, dec1_g, dec1_be,
           dec2_W, dec2_b, dec2_g, dec2_be):
    e0w, e0b = _fold(enc0_W, enc0_b, enc0_g, enc0_be)
    e1w, e1b = _fold(enc1_W, enc1_b, enc1_g, enc1_be)
    e2w, e2b = _fold(enc2_W, enc2_b, enc2_g, enc2_be)
    d0w, d0b = _fold(dec0_W, dec0_b, dec0_g, dec0_be)
    d1w, d1b = _fold(dec1_W, dec1_b, dec1_g, dec1_be)
    d2w, d2b = _fold(dec2_W, dec2_b, dec2_g, dec2_be)
    wl1, wr1, b1 = si_Wl.T, si_Wr.T, si_b[None, :]
    wl2, wr2, b2 = so_Wl.T, so_Wr.T, so_b[None, :]

    src, dst = edge_index[0], edge_index[1]
    src_p = jnp.concatenate([src, jnp.zeros((E_PAD - N_EDGES,), jnp.int32)])
    dst_p = jnp.concatenate([dst, jnp.full((E_PAD - N_EDGES,), N_NODES, jnp.int32)])
    tail = TOT_CHUNKS - SCAT_CHUNKS
    src_g = jnp.concatenate(
        [src_p.reshape(NW, SCAT_CHUNKS, CHUNK),
         jnp.zeros((NW, tail, CHUNK), jnp.int32)], axis=1).reshape(NW * TOT_CHUNKS, CHUNK)
    dst_g = jnp.concatenate(
        [dst_p.reshape(NW, SCAT_CHUNKS, CHUNK),
         jnp.full((NW, tail, CHUNK), N_NODES, jnp.int32)], axis=1).reshape(NW * TOT_CHUNKS, CHUNK)

    h0 = pl.pallas_call(
        _enc_body,
        grid=(_GRID,),
        in_specs=[_row_block(128),
                  _full_block((128, 32)), _full_block((1, 32)),
                  _full_block((32, 32)), _full_block((1, 32)),
                  _full_block((32, 16)), _full_block((1, 16))],
        out_specs=_row_block(16),
        out_shape=jax.ShapeDtypeStruct((N_NODES, 16), jnp.float32),
    )(x, e0w, e0b, e1w, e1b, e2w, e2b)

    dst_f = jnp.concatenate(
        [dst_p.reshape(NW, SCAT_CHUNKS * CHUNK),
         jnp.full((NW, (TOT_CHUNKS - SCAT_CHUNKS) * CHUNK), N_NODES, jnp.int32)],
        axis=1)
    zrows = jnp.zeros((RPT, 16), jnp.float32)

    p0, p1, oc0, oc1 = _sc_agg_cnt(src_g, dst_g, dst_f, zrows, h0)
    c0 = oc0.reshape(R, 1)
    c1 = oc1.reshape(R, 1)

    ptab, h1r, rinv = pl.pallas_call(
        _mid_body,
        grid=(_GRID,),
        in_specs=[_row_block(16), _row_block(16),
                  _row_block(1), _row_block(1), _row_block(16),
                  _full_block((16, 32)), _full_block((16, 32)), _full_block((1, 32)),
                  _full_block((32, 16)), _full_block((32, 16)), _full_block((1, 16))],
        out_specs=[_row_block(16), _row_block(16), _row_block(16)],
        out_shape=[jax.ShapeDtypeStruct((N_NODES, 16), jnp.float32),
                   jax.ShapeDtypeStruct((N_NODES, 16), jnp.float32),
                   jax.ShapeDtypeStruct((N_NODES, 16), jnp.float32)],
    )(p0, p1, c0, c1, h0, wl1, wr1, b1, wl2, wr2, b2)

    q0, q1 = _sc_agg16(src_g, dst_g, dst_f, zrows, ptab)

    out = pl.pallas_call(
        _dec_body,
        grid=(_GRID,),
        in_specs=[_row_block(16), _row_block(16), _row_block(16), _row_block(16),
                  _full_block((16, 32)), _full_block((1, 32)),
                  _full_block((32, 32)), _full_block((1, 32)),
                  _full_block((32, 128)), _full_block((1, 128))],
        out_specs=_row_block(128),
        out_shape=jax.ShapeDtypeStruct((N_NODES, 128), jnp.float32),
    )(q0, q1, h1r, rinv, d0w, d0b, d1w, d1b, d2w, d2b)

    return out


# speedup vs baseline: 2.4340x; 1.1124x over previous
import functools

import jax
import jax.numpy as jnp
from jax import lax
from jax.experimental import pallas as pl
from jax.experimental.pallas import tpu as pltpu
from jax.experimental.pallas import tpu_sc as plsc

N_NODES = 10000
N_EDGES = 320000

NC, NS = 2, 16
NW = NC * NS
CHUNK = 128
A_CHUNKS = 96
B_CHUNKS = 64
TAILC = 8
ABLK = A_CHUNKS + TAILC
BBLK = B_CHUNKS + TAILC
G_ROWS = NS * ABLK + NS * BBLK + (A_CHUNKS - B_CHUNKS)
E_PAD = NS * (A_CHUNKS + B_CHUNKS) * CHUNK
R = 10240
RPT = R // NS

_SELU_SCALE = 1.0507009873554805
_SELU_ALPHA = 1.6732632423543772


def _selu(x):
    return _SELU_SCALE * jnp.where(x > 0, x, _SELU_ALPHA * (jnp.exp(x) - 1.0))


def _make_sc_agg(with_counts):
    W = 16
    mesh = plsc.VectorSubcoreMesh(core_axis_name="c", subcore_axis_name="s")
    out_types = [
        jax.ShapeDtypeStruct((R, W), jnp.float32),
        jax.ShapeDtypeStruct((R, W), jnp.float32),
    ]
    scratch = [
        pltpu.VMEM((ABLK, CHUNK), jnp.int32),
        pltpu.VMEM((ABLK, CHUNK), jnp.int32),
        pltpu.VMEM((CHUNK, W), jnp.float32),
        pltpu.VMEM((CHUNK, W), jnp.float32),
        pltpu.VMEM((RPT, W), jnp.float32),
        pltpu.VMEM_SHARED((R, W), jnp.float32),
        pltpu.SemaphoreType.DMA,
        pltpu.SemaphoreType.DMA,
    ]
    if with_counts:
        out_types += [
            jax.ShapeDtypeStruct((R,), jnp.float32),
            jax.ShapeDtypeStruct((R,), jnp.float32),
        ]
        scratch += [
            pltpu.VMEM((A_CHUNKS * CHUNK,), jnp.int32),
            pltpu.VMEM((R,), jnp.float32),
            pltpu.VMEM((RPT,), jnp.float32),
            pltpu.VMEM_SHARED((NS * R,), jnp.float32),
        ]

    @functools.partial(
        pl.kernel,
        out_type=tuple(out_types),
        mesh=mesh,
        scratch_types=scratch,
        compiler_params=pltpu.CompilerParams(use_tc_tiling_on_sc=False,
                                             needs_layout_passes=not with_counts),
    )
    def sc_agg(src_hbm, dst_hbm, dstf_hbm, zeros_hbm, tab_hbm, *rest):
        if with_counts:
            (out0, out1, oc0, oc1,
             idx_s, idx_d, buf0, buf1, obuf, acc, sem0, sem1,
             idx_f, cnt, cseg, acc_c) = rest
        else:
            (out0, out1,
             idx_s, idx_d, buf0, buf1, obuf, acc, sem0, sem1) = rest
        c = lax.axis_index("c")
        s = lax.axis_index("s")
        base = jnp.where(c == 0, s * ABLK, NS * ABLK + s * BBLK)
        fbase = jnp.where(c == 0, s * A_CHUNKS,
                          NS * A_CHUNKS + s * B_CHUNKS) * CHUNK
        ngrp = jnp.where(c == 0, A_CHUNKS // 2, B_CHUNKS // 2)

        pltpu.sync_copy(src_hbm.at[pl.ds(base, ABLK)], idx_s)
        pltpu.sync_copy(dst_hbm.at[pl.ds(base, ABLK)], idx_d)

        pltpu.sync_copy(zeros_hbm, obuf)
        pltpu.sync_copy(obuf, acc.at[pl.ds(s * RPT, RPT)])

        zv = jnp.zeros((16,), jnp.float32)
        ov = jnp.ones((16,), jnp.float32)
        if with_counts:
            pltpu.sync_copy(dstf_hbm.at[pl.ds(fbase, A_CHUNKS * CHUNK)], idx_f)

            def zcrow(i, carry):
                cnt[pl.ds(i * 16, 16)] = zv
                return carry

            lax.fori_loop(0, R // 16, zcrow, 0)
        plsc.subcore_barrier()

        def gstart(j, buf, sem):
            return pltpu.async_copy(tab_hbm.at[idx_s.at[j]], buf, sem)

        gstart(0, buf0, sem0)
        gstart(1, buf1, sem1)

        def body(jj, carry):
            j = jj * 2
            pltpu.make_async_copy(tab_hbm.at[idx_s.at[j]], buf0, sem0).wait()
            pltpu.sync_copy(buf0, acc.at[idx_d.at[j]], add=True)
            gstart(j + 2, buf0, sem0)
            if with_counts:
                for k in range(2 * CHUNK // 16):
                    dv = idx_f[pl.ds(jj * 2 * CHUNK + k * 16, 16)]
                    plsc.addupdate_scatter(cnt, [dv], ov)
            pltpu.make_async_copy(tab_hbm.at[idx_s.at[j + 1]], buf1, sem1).wait()
            pltpu.sync_copy(buf1, acc.at[idx_d.at[j + 1]], add=True)
            gstart(j + 3, buf1, sem1)
            return carry

        lax.fori_loop(0, ngrp, body, 0)

        pltpu.make_async_copy(tab_hbm.at[idx_s.at[0]], buf0, sem0).wait()
        pltpu.make_async_copy(tab_hbm.at[idx_s.at[0]], buf1, sem1).wait()

        if with_counts:
            pltpu.sync_copy(cnt, acc_c.at[pl.ds(s * R, R)])

        plsc.subcore_barrier()

        pltpu.sync_copy(acc.at[pl.ds(s * RPT, RPT)], obuf)

        @pl.when(c == 0)
        def _():
            pltpu.sync_copy(obuf, out0.at[pl.ds(s * RPT, RPT)])

        @pl.when(c == 1)
        def _():
            pltpu.sync_copy(obuf, out1.at[pl.ds(s * RPT, RPT)])

        if with_counts:
            def gslice(t, carry):
                pltpu.sync_copy(acc_c.at[pl.ds(t * R + s * RPT, RPT)],
                                cnt.at[pl.ds(t * RPT, RPT)])
                return carry

            lax.fori_loop(0, NS, gslice, 0)

            def redrow(v, carry):
                a = cnt[pl.ds(v * 16, 16)]
                for t in range(1, NS):
                    a = a + cnt[pl.ds(t * RPT + v * 16, 16)]
                cseg[pl.ds(v * 16, 16)] = a
                return carry

            lax.fori_loop(0, RPT // 16, redrow, 0)

            @pl.when(c == 0)
            def _():
                pltpu.sync_copy(cseg, oc0.at[pl.ds(s * RPT, RPT)])

            @pl.when(c == 1)
            def _():
                pltpu.sync_copy(cseg, oc1.at[pl.ds(s * RPT, RPT)])

    return sc_agg


_sc_agg_cnt = _make_sc_agg(True)
_sc_agg16 = _make_sc_agg(False)


_BLK = 2000
_GRID = N_NODES // _BLK


def _row_block(width):
    return pl.BlockSpec((_BLK, width), lambda i: (i, 0))


def _full_block(shape):
    return pl.BlockSpec(shape, lambda i: (0,) * len(shape))


def _enc_body(x_ref, w0, b0, w1, b1, w2, b2, out_ref):
    h = _selu(jnp.dot(x_ref[...], w0[...], preferred_element_type=jnp.float32) + b0[...])
    h = _selu(jnp.dot(h, w1[...], preferred_element_type=jnp.float32) + b1[...])
    out_ref[...] = _selu(jnp.dot(h, w2[...], preferred_element_type=jnp.float32) + b2[...])


def _mid_body(p0, p1, c0, c1, h0ref, wl1, wr1, b1, wl2, wr2, b2,
              ptab, h1r, rinv_out):
    ssum = p0[...] + p1[...]
    cnt = c0[...] + c1[...]
    rinv = 1.0 / jnp.maximum(cnt, 1.0)
    agg0 = ssum * rinv
    h0 = h0ref[...]
    h1 = (jnp.dot(agg0, wl1[...], preferred_element_type=jnp.float32)
          + jnp.dot(h0, wr1[...], preferred_element_type=jnp.float32) + b1[...])
    ptab[...] = jnp.dot(h1, wl2[...], preferred_element_type=jnp.float32)
    h1r[...] = jnp.dot(h1, wr2[...], preferred_element_type=jnp.float32) + b2[...]
    rinv_out[...] = jnp.broadcast_to(rinv, (_BLK, 16))


def _dec_body(q0, q1, h1r, rinv, w0, b0, w1, b1, w2, b2, out_ref):
    h2 = (q0[...] + q1[...]) * rinv[...] + h1r[...]
    h = _selu(jnp.dot(h2, w0[...], preferred_element_type=jnp.float32) + b0[...])
    h = _selu(jnp.dot(h, w1[...], preferred_element_type=jnp.float32) + b1[...])
    h = _selu(jnp.dot(h, w2[...], preferred_element_type=jnp.float32) + b2[...])
    out_ref[...] = h


def _fold(W, b, g, be):
    Wf = (W * g[:, None]).T
    bf = (b * g + be)[None, :]
    return Wf, bf


def kernel(x, edge_index,
           enc0_W, enc0_b, enc0_g, enc0_be,
           enc1_W, enc1_b, enc1_g, enc1_be,
           enc2_W, enc2_b, enc2_g, enc2_be,
           si_Wl, si_Wr, si_b,
           so_Wl, so_Wr, so_b,
           dec0_W, dec0_b, dec0_g, dec0_be,
           dec1_W, dec1_b, dec1_g, dec1_be,
           dec2_W, dec2_b, dec2_g, dec2_be):
    e0w, e0b = _fold(enc0_W, enc0_b, enc0_g, enc0_be)
    e1w, e1b = _fold(enc1_W, enc1_b, enc1_g, enc1_be)
    e2w, e2b = _fold(enc2_W, enc2_b, enc2_g, enc2_be)
    d0w, d0b = _fold(dec0_W, dec0_b, dec0_g, dec0_be)
    d1w, d1b = _fold(dec1_W, dec1_b, dec1_g, dec1_be)
    d2w, d2b = _fold(dec2_W, dec2_b, dec2_g, dec2_be)
    wl1, wr1, b1 = si_Wl.T, si_Wr.T, si_b[None, :]
    wl2, wr2, b2 = so_Wl.T, so_Wr.T, so_b[None, :]

    src, dst = edge_index[0], edge_index[1]
    src_p = jnp.concatenate([src, jnp.zeros((E_PAD - N_EDGES,), jnp.int32)])
    dst_p = jnp.concatenate([dst, jnp.full((E_PAD - N_EDGES,), N_NODES, jnp.int32)])
    nA = NS * A_CHUNKS * CHUNK

    def _blocks(flat, fill):
        a = flat[:nA].reshape(NS, A_CHUNKS, CHUNK)
        b = flat[nA:].reshape(NS, B_CHUNKS, CHUNK)
        tl = jnp.full((NS, TAILC, CHUNK), fill, jnp.int32)
        return jnp.concatenate(
            [jnp.concatenate([a, tl], axis=1).reshape(NS * ABLK, CHUNK),
             jnp.concatenate([b, tl], axis=1).reshape(NS * BBLK, CHUNK),
             jnp.full((A_CHUNKS - B_CHUNKS, CHUNK), fill, jnp.int32)], axis=0)

    src_g = _blocks(src_p, 0)
    dst_g = _blocks(dst_p, N_NODES)

    h0 = pl.pallas_call(
        _enc_body,
        grid=(_GRID,),
        in_specs=[_row_block(128),
                  _full_block((128, 32)), _full_block((1, 32)),
                  _full_block((32, 32)), _full_block((1, 32)),
                  _full_block((32, 16)), _full_block((1, 16))],
        out_specs=_row_block(16),
        out_shape=jax.ShapeDtypeStruct((N_NODES, 16), jnp.float32),
    )(x, e0w, e0b, e1w, e1b, e2w, e2b)

    dst_f = jnp.concatenate(
        [dst_p, jnp.full(((A_CHUNKS - B_CHUNKS) * CHUNK,), N_NODES, jnp.int32)])
    zrows = jnp.zeros((RPT, 16), jnp.float32)

    p0, p1, oc0, oc1 = _sc_agg_cnt(src_g, dst_g, dst_f, zrows, h0)
    c0 = oc0.reshape(R, 1)
    c1 = oc1.reshape(R, 1)

    ptab, h1r, rinv = pl.pallas_call(
        _mid_body,
        grid=(_GRID,),
        in_specs=[_row_block(16), _row_block(16),
                  _row_block(1), _row_block(1), _row_block(16),
                  _full_block((16, 32)), _full_block((16, 32)), _full_block((1, 32)),
                  _full_block((32, 16)), _full_block((32, 16)), _full_block((1, 16))],
        out_specs=[_row_block(16), _row_block(16), _row_block(16)],
        out_shape=[jax.ShapeDtypeStruct((N_NODES, 16), jnp.float32),
                   jax.ShapeDtypeStruct((N_NODES, 16), jnp.float32),
                   jax.ShapeDtypeStruct((N_NODES, 16), jnp.float32)],
    )(p0, p1, c0, c1, h0, wl1, wr1, b1, wl2, wr2, b2)

    q0, q1 = _sc_agg16(src_g, dst_g, dst_f, zrows, ptab)

    out = pl.pallas_call(
        _dec_body,
        grid=(_GRID,),
        in_specs=[_row_block(16), _row_block(16), _row_block(16), _row_block(16),
                  _full_block((16, 32)), _full_block((1, 32)),
                  _full_block((32, 32)), _full_block((1, 32)),
                  _full_block((32, 128)), _full_block((1, 128))],
        out_specs=_row_block(128),
        out_shape=jax.ShapeDtypeStruct((N_NODES, 128), jnp.float32),
    )(q0, q1, h1r, rinv, d0w, d0b, d1w, d1b, d2w, d2b)

    return out
